# baseline (speedup 1.0000x reference)
import functools

import jax
import jax.numpy as jnp
from jax import lax
from jax.experimental import pallas as pl
from jax.experimental.pallas import tpu as pltpu
from jax.experimental.pallas import tpu_sc as plsc

_NC = 2
_NS = 16
_NW = _NC * _NS
_CH = 128


def _build_deg(n_chunks_w, n_pad):
    mesh = plsc.VectorSubcoreMesh(core_axis_name="c", subcore_axis_name="s")
    rpt = n_pad // _NS

    @functools.partial(
        pl.kernel,
        mesh=mesh,
        out_type=jax.ShapeDtypeStruct((_NC, n_pad, 16), jnp.float32),
        scratch_types=[
            pltpu.VMEM((n_chunks_w, _CH), jnp.int32),
            pltpu.VMEM((_CH, 16), jnp.float32),
            pltpu.VMEM_SHARED((n_pad, 16), jnp.float32),
        ],
    )
    def degk(dst_hbm, ones_hbm, zeros_hbm, out_hbm, didx, ones, acc):
        c = lax.axis_index("c")
        s = lax.axis_index("s")
        w = c * _NS + s

        r0 = s * rpt
        pltpu.sync_copy(zeros_hbm.at[pl.ds(r0, rpt)], acc.at[pl.ds(r0, rpt)])
        pltpu.sync_copy(ones_hbm, ones)
        pltpu.sync_copy(dst_hbm.at[pl.ds(w * n_chunks_w, n_chunks_w)], didx)
        plsc.subcore_barrier()

        def body(i, _):
            pltpu.sync_copy(ones, acc.at[didx.at[i]], add=True)
            return 0

        lax.fori_loop(0, n_chunks_w, body, 0)
        plsc.subcore_barrier()
        pltpu.sync_copy(acc.at[pl.ds(r0, rpt)], out_hbm.at[c, pl.ds(r0, rpt)])

    return degk


def _build_prop(n_chunks_w, n_pad, d):
    mesh = plsc.VectorSubcoreMesh(core_axis_name="c", subcore_axis_name="s")
    rpt = n_pad // _NS
    nph = 2
    npc = n_chunks_w // nph

    @functools.partial(
        pl.kernel,
        mesh=mesh,
        out_type=jax.ShapeDtypeStruct((_NC, n_pad, d), jnp.float32),
        scratch_types=[
            pltpu.VMEM((npc + 1, _CH), jnp.int32),
            pltpu.VMEM((npc, _CH), jnp.int32),
            pltpu.VMEM((2, _CH, d), jnp.float32),
            pltpu.VMEM_SHARED((n_pad, d), jnp.float32),
            pltpu.SemaphoreType.DMA,
            pltpu.SemaphoreType.DMA,
        ],
    )
    def prop(src_hbm, dst_hbm, y_hbm, zeros_hbm, out_hbm, sidx, didx, rows,
             acc, sem0, sem1):
        c = lax.axis_index("c")
        s = lax.axis_index("s")
        w = c * _NS + s
        sems = (sem0, sem1)

        r0 = s * rpt
        pltpu.sync_copy(zeros_hbm.at[pl.ds(r0, rpt)], acc.at[pl.ds(r0, rpt)])
        plsc.subcore_barrier()

        for p in range(nph):
            cr0 = w * n_chunks_w + p * npc
            pltpu.sync_copy(src_hbm.at[pl.ds(cr0, npc)],
                            sidx.at[pl.ds(0, npc)])
            pltpu.sync_copy(dst_hbm.at[pl.ds(cr0, npc)], didx)

            def body(j, _):
                pltpu.async_copy(y_hbm.at[sidx.at[j]], rows.at[0], sem0).wait()
                pltpu.sync_copy(rows.at[0], acc.at[didx.at[j]], add=True)
                return 0

            lax.fori_loop(0, npc, body, 0)
        plsc.subcore_barrier()
        pltpu.sync_copy(acc.at[pl.ds(r0, rpt)], out_hbm.at[c, pl.ds(r0, rpt)])

    return prop


def _deg_from_cnt(cnt_ref):
    return 1.0 + jnp.sum(cnt_ref[...], axis=(0, 2))[:, None] * (1.0 / 16.0)


def _tc1_body(cnt_ref, x_ref, y_ref):
    y_ref[...] = x_ref[...] * lax.rsqrt(_deg_from_cnt(cnt_ref))


def _tc2_body(cnt_ref, y0_ref, p_ref, o_ref):
    o_ref[...] = (y0_ref[...] + p_ref[0] + p_ref[1]) / _deg_from_cnt(cnt_ref)


def _tc3_body(cnt_ref, y1_ref, q_ref, w_ref, b_ref, o_ref):
    h2 = (y1_ref[...] + q_ref[0] + q_ref[1]) * lax.rsqrt(_deg_from_cnt(cnt_ref))
    o = lax.dot_general(
        h2, w_ref[...], (((1,), (0,)), ((), ())),
        precision=lax.Precision.HIGHEST,
        preferred_element_type=jnp.float32,
    ) + b_ref[...]
    m = jnp.max(o, axis=1, keepdims=True)
    lse = m + jnp.log(jnp.sum(jnp.exp(o - m), axis=1, keepdims=True))
    o_ref[...] = o - lse


def kernel(x, edge_index, W, b):
    n, d = x.shape
    e = edge_index.shape[1]

    n_chunks_w = -(-e // (_NW * _CH * 8)) * 8
    ep = _NW * _CH * n_chunks_w
    pad = ep - e
    src = jnp.concatenate(
        [edge_index[0], jnp.zeros((pad,), edge_index.dtype)])
    dst = jnp.concatenate(
        [edge_index[1], jnp.full((pad,), n, edge_index.dtype)])
    src2 = src.reshape(ep // _CH, _CH)
    dst2 = dst.reshape(ep // _CH, _CH)

    br = 1024
    n2 = -(-n // br) * br
    if n2 == n:
        n2 += br
    x2 = jnp.concatenate([x, jnp.zeros((n2 - n, d), x.dtype)])

    degk = _build_deg(n_chunks_w, n2)
    prop = _build_prop(n_chunks_w, n2, d)

    zrows = jnp.zeros((n2, d), jnp.float32)
    ones16 = jnp.ones((_CH, 16), jnp.float32)
    zcnt = jnp.zeros((n2, 16), jnp.float32)

    cnt = degk(dst2, ones16, zcnt)

    grid = (n2 // br,)
    cnt_spec = pl.BlockSpec((2, br, 16), lambda i: (0, i, 0))
    row_spec = pl.BlockSpec((br, d), lambda i: (i, 0))
    par_spec = pl.BlockSpec((2, br, d), lambda i: (0, i, 0))
    out_sds = jax.ShapeDtypeStruct((n2, d), jnp.float32)

    y0 = pl.pallas_call(
        _tc1_body,
        grid=grid,
        in_specs=[cnt_spec, row_spec],
        out_specs=row_spec,
        out_shape=out_sds,
    )(cnt, x2)

    p = prop(src2, dst2, y0, zrows)

    y1 = pl.pallas_call(
        _tc2_body,
        grid=grid,
        in_specs=[cnt_spec, row_spec, par_spec],
        out_specs=row_spec,
        out_shape=out_sds,
    )(cnt, y0, p)

    q = prop(src2, dst2, y1, zrows)

    out = pl.pallas_call(
        _tc3_body,
        grid=grid,
        in_specs=[
            cnt_spec,
            row_spec,
            par_spec,
            pl.BlockSpec((d, d), lambda i: (0, 0)),
            pl.BlockSpec((1, d), lambda i: (0, 0)),
        ],
        out_specs=row_spec,
        out_shape=out_sds,
    )(cnt, y1, q, W, b.reshape(1, d))

    return out[:n]


# speedup vs baseline: 1.0267x; 1.0267x over previous
import functools

import jax
import jax.numpy as jnp
from jax import lax
from jax.experimental import pallas as pl
from jax.experimental.pallas import tpu as pltpu
from jax.experimental.pallas import tpu_sc as plsc

_NC = 2
_NS = 16
_NW = _NC * _NS
_CH = 128


def _build_deg(n_chunks_w, n_pad):
    mesh = plsc.VectorSubcoreMesh(core_axis_name="c", subcore_axis_name="s")
    rpt = n_pad // _NS

    @functools.partial(
        pl.kernel,
        mesh=mesh,
        out_type=jax.ShapeDtypeStruct((_NC, n_pad, 16), jnp.float32),
        scratch_types=[
            pltpu.VMEM((n_chunks_w, _CH), jnp.int32),
            pltpu.VMEM((_CH, 16), jnp.float32),
            pltpu.VMEM_SHARED((n_pad, 16), jnp.float32),
        ],
    )
    def degk(dst_hbm, ones_hbm, zeros_hbm, out_hbm, didx, ones, acc):
        c = lax.axis_index("c")
        s = lax.axis_index("s")
        w = c * _NS + s

        r0 = s * rpt
        pltpu.sync_copy(zeros_hbm.at[pl.ds(r0, rpt)], acc.at[pl.ds(r0, rpt)])
        pltpu.sync_copy(ones_hbm, ones)
        pltpu.sync_copy(dst_hbm.at[pl.ds(w * n_chunks_w, n_chunks_w)], didx)
        plsc.subcore_barrier()

        def body(i, _):
            pltpu.sync_copy(ones, acc.at[didx.at[i]], add=True)
            return 0

        lax.fori_loop(0, n_chunks_w, body, 0)
        plsc.subcore_barrier()
        pltpu.sync_copy(acc.at[pl.ds(r0, rpt)], out_hbm.at[c, pl.ds(r0, rpt)])

    return degk


def _build_prop(n_chunks_w, n_pad, d):
    mesh = plsc.VectorSubcoreMesh(core_axis_name="c", subcore_axis_name="s")
    rpt = n_pad // _NS
    nph = 2
    npc = n_chunks_w // nph

    @functools.partial(
        pl.kernel,
        mesh=mesh,
        out_type=jax.ShapeDtypeStruct((_NC, n_pad, d), jnp.float32),
        scratch_types=[
            pltpu.VMEM((npc + 1, _CH), jnp.int32),
            pltpu.VMEM((npc, _CH), jnp.int32),
            pltpu.VMEM((2, _CH, d), jnp.float32),
            pltpu.VMEM_SHARED((n_pad, d), jnp.float32),
            pltpu.SemaphoreType.DMA,
            pltpu.SemaphoreType.DMA,
        ],
    )
    def prop(src_hbm, dst_hbm, y_hbm, zeros_hbm, out_hbm, sidx, didx, rows,
             acc, sem0, sem1):
        c = lax.axis_index("c")
        s = lax.axis_index("s")
        w = c * _NS + s

        r0 = s * rpt
        pltpu.sync_copy(zeros_hbm.at[pl.ds(r0, rpt)], acc.at[pl.ds(r0, rpt)])
        plsc.subcore_barrier()

        for p in range(nph):
            cr0 = w * n_chunks_w + p * npc
            pltpu.sync_copy(src_hbm.at[pl.ds(cr0, npc)],
                            sidx.at[pl.ds(0, npc)])
            pltpu.sync_copy(dst_hbm.at[pl.ds(cr0, npc)], didx)

            def body(j2, _):
                j = j2 * 2
                cp0 = pltpu.async_copy(y_hbm.at[sidx.at[j]], rows.at[0], sem0)
                cp1 = pltpu.async_copy(y_hbm.at[sidx.at[j + 1]], rows.at[1],
                                       sem0)
                cp0.wait()
                cp1.wait()
                pltpu.sync_copy(rows.at[0], acc.at[didx.at[j]], add=True)
                pltpu.sync_copy(rows.at[1], acc.at[didx.at[j + 1]], add=True)
                return 0

            lax.fori_loop(0, npc // 2, body, 0)
        plsc.subcore_barrier()
        pltpu.sync_copy(acc.at[pl.ds(r0, rpt)], out_hbm.at[c, pl.ds(r0, rpt)])

    return prop


def _deg_from_cnt(cnt_ref):
    return 1.0 + jnp.sum(cnt_ref[...], axis=(0, 2))[:, None] * (1.0 / 16.0)


def _tc1_body(cnt_ref, x_ref, y_ref):
    y_ref[...] = x_ref[...] * lax.rsqrt(_deg_from_cnt(cnt_ref))


def _tc2_body(cnt_ref, y0_ref, p_ref, o_ref):
    o_ref[...] = (y0_ref[...] + p_ref[0] + p_ref[1]) / _deg_from_cnt(cnt_ref)


def _tc3_body(cnt_ref, y1_ref, q_ref, w_ref, b_ref, o_ref):
    h2 = (y1_ref[...] + q_ref[0] + q_ref[1]) * lax.rsqrt(_deg_from_cnt(cnt_ref))
    o = lax.dot_general(
        h2, w_ref[...], (((1,), (0,)), ((), ())),
        precision=lax.Precision.HIGHEST,
        preferred_element_type=jnp.float32,
    ) + b_ref[...]
    m = jnp.max(o, axis=1, keepdims=True)
    lse = m + jnp.log(jnp.sum(jnp.exp(o - m), axis=1, keepdims=True))
    o_ref[...] = o - lse


def kernel(x, edge_index, W, b):
    n, d = x.shape
    e = edge_index.shape[1]

    n_chunks_w = -(-e // (_NW * _CH * 8)) * 8
    ep = _NW * _CH * n_chunks_w
    pad = ep - e
    src = jnp.concatenate(
        [edge_index[0], jnp.zeros((pad,), edge_index.dtype)])
    dst = jnp.concatenate(
        [edge_index[1], jnp.full((pad,), n, edge_index.dtype)])
    src2 = src.reshape(ep // _CH, _CH)
    dst2 = dst.reshape(ep // _CH, _CH)

    br = 1024
    n2 = -(-n // br) * br
    if n2 == n:
        n2 += br
    x2 = jnp.concatenate([x, jnp.zeros((n2 - n, d), x.dtype)])

    degk = _build_deg(n_chunks_w, n2)
    prop = _build_prop(n_chunks_w, n2, d)

    zrows = jnp.zeros((n2, d), jnp.float32)
    ones16 = jnp.ones((_CH, 16), jnp.float32)
    zcnt = jnp.zeros((n2, 16), jnp.float32)

    cnt = degk(dst2, ones16, zcnt)

    grid = (n2 // br,)
    cnt_spec = pl.BlockSpec((2, br, 16), lambda i: (0, i, 0))
    row_spec = pl.BlockSpec((br, d), lambda i: (i, 0))
    par_spec = pl.BlockSpec((2, br, d), lambda i: (0, i, 0))
    out_sds = jax.ShapeDtypeStruct((n2, d), jnp.float32)

    y0 = pl.pallas_call(
        _tc1_body,
        grid=grid,
        in_specs=[cnt_spec, row_spec],
        out_specs=row_spec,
        out_shape=out_sds,
    )(cnt, x2)

    p = prop(src2, dst2, y0, zrows)

    y1 = pl.pallas_call(
        _tc2_body,
        grid=grid,
        in_specs=[cnt_spec, row_spec, par_spec],
        out_specs=row_spec,
        out_shape=out_sds,
    )(cnt, y0, p)

    q = prop(src2, dst2, y1, zrows)

    out = pl.pallas_call(
        _tc3_body,
        grid=grid,
        in_specs=[
            cnt_spec,
            row_spec,
            par_spec,
            pl.BlockSpec((d, d), lambda i: (0, 0)),
            pl.BlockSpec((1, d), lambda i: (0, 0)),
        ],
        out_specs=row_spec,
        out_shape=out_sds,
    )(cnt, y1, q, W, b.reshape(1, d))

    return out[:n]
